# baseline (speedup 1.0000x reference)
import jax
import jax.numpy as jnp
from jax import lax
from jax.experimental import pallas as pl
from jax.experimental.pallas import tpu as pltpu

N = 5000
NPAD = 5120
NBLK = 8
NSUB = 640
NUM_CLASSES = 21
C = NUM_CLASSES - 1
NUM_DET = 100
SCORE_T = 0.05
NMS_T = 0.5
IMG_W = 1333.0
IMG_H = 800.0
MIN_SIZE = 1.0
LOG_MAX = 4.135166556742356
OFF_STEP = IMG_W + 2.0
NEG_INF = float("-inf")

_INTERPRET = False


def _precompute_body(logit_ref, d4_ref, prop_ref, nbx1_ref, nby1_ref,
                     nbx2_ref, nby2_ref, work_ref, s0_ref):
    logit = logit_ref[...]
    m = jnp.max(logit, axis=0, keepdims=True)
    e = jnp.exp(logit - m)
    ssum = jnp.sum(e, axis=0, keepdims=True)
    scores_all = e / ssum
    scores = scores_all[1:, :]

    px1 = prop_ref[0, :][None, :]
    py1 = prop_ref[1, :][None, :]
    px2 = prop_ref[2, :][None, :]
    py2 = prop_ref[3, :][None, :]
    widths = px2 - px1
    heights = py2 - py1
    ctr_x = px1 + 0.5 * widths
    ctr_y = py1 + 0.5 * heights

    dx = d4_ref[0] / 10.0
    dy = d4_ref[1] / 10.0
    dw = jnp.minimum(d4_ref[2] / 5.0, LOG_MAX)
    dh = jnp.minimum(d4_ref[3] / 5.0, LOG_MAX)

    pcx = dx * widths + ctr_x
    pcy = dy * heights + ctr_y
    pw = jnp.exp(dw) * widths
    ph = jnp.exp(dh) * heights

    x1 = jnp.clip(pcx - 0.5 * pw, 0.0, IMG_W)
    y1 = jnp.clip(pcy - 0.5 * ph, 0.0, IMG_H)
    x2 = jnp.clip(pcx + 0.5 * pw, 0.0, IMG_W)
    y2 = jnp.clip(pcy + 0.5 * ph, 0.0, IMG_H)

    cls_iota = lax.broadcasted_iota(jnp.int32, (C, NPAD), 0).astype(jnp.float32)
    offset = (cls_iota + 1.0) * OFF_STEP
    nbx1 = x1 + offset
    nby1 = y1 + offset
    nbx2 = x2 + offset
    nby2 = y2 + offset

    ws = x2 - x1
    hs = y2 - y1
    lane = lax.broadcasted_iota(jnp.int32, (C, NPAD), 1)
    valid = (scores > SCORE_T) & (ws >= MIN_SIZE) & (hs >= MIN_SIZE) \
        & (lane < N)
    work = jnp.where(valid, scores, NEG_INF)

    nbx1_ref[...] = nbx1
    nby1_ref[...] = nby1
    nbx2_ref[...] = nbx2
    nby2_ref[...] = nby2
    work_ref[...] = work
    s0_ref[...] = scores[0:1, 0:1]


def _nms_body(nbx1_ref, nby1_ref, nbx2_ref, nby2_ref, work_in_ref, s0_ref,
              out_ref, work_ref):
    work_ref[...] = work_in_ref[...]
    s0v = s0_ref[...]

    lane128 = lax.broadcasted_iota(jnp.int32, (1, 128), 1)
    vec = jnp.full((1, 128), NEG_INF, dtype=jnp.float32)
    for c in range(C):
        mc = jnp.max(work_ref[c])
        vec = jnp.where(lane128 == c, mc, vec)

    sub_iota = lax.broadcasted_iota(jnp.int32, (NBLK, NSUB), 0)
    lane_iota = lax.broadcasted_iota(jnp.int32, (NBLK, NSUB), 1)
    flat_local = sub_iota * NSUB + lane_iota
    liota = lax.broadcasted_iota(jnp.int32, (1, 8), 1)

    def body(t, carry):
        vec, mcv, clsv_p, restv, cls_rv = carry
        vv = jnp.maximum(mcv, restv)
        is_fbv = vv == NEG_INF
        clsv = jnp.where(mcv >= restv, clsv_p, cls_rv)
        clsv = jnp.where(is_fbv, 0, clsv)
        cls = clsv[0, 0]

        w_c = work_ref[cls]
        eq = (w_c == vv) & (~is_fbv | (flat_local == 0))

        nx1 = nbx1_ref[cls]
        ny1 = nby1_ref[cls]
        nx2 = nbx2_ref[cls]
        ny2 = nby2_ref[cls]
        bx1 = jnp.sum(jnp.where(eq, nx1, 0.0), keepdims=True)
        by1 = jnp.sum(jnp.where(eq, ny1, 0.0), keepdims=True)
        bx2 = jnp.sum(jnp.where(eq, nx2, 0.0), keepdims=True)
        by2 = jnp.sum(jnp.where(eq, ny2, 0.0), keepdims=True)
        ba = (bx2 - bx1) * (by2 - by1)

        xx1 = jnp.maximum(nx1, bx1)
        yy1 = jnp.maximum(ny1, by1)
        xx2 = jnp.minimum(nx2, bx2)
        yy2 = jnp.minimum(ny2, by2)
        areas = (nx2 - nx1) * (ny2 - ny1)
        inter = jnp.maximum(xx2 - xx1, 0.0) * jnp.maximum(yy2 - yy1, 0.0)
        iou = inter / (areas + ba - inter + 1e-9)
        new_w = jnp.where((iou > NMS_T) | eq, NEG_INF, w_c)
        work_ref[cls] = new_w

        vec_m = jnp.where(lane128 == clsv, NEG_INF, vec)
        rest_n = jnp.max(vec_m, keepdims=True)
        cls_rn = jnp.min(jnp.where(vec_m == rest_n, lane128, 127),
                         keepdims=True)

        mc = jnp.max(new_w, keepdims=True).reshape(1, 1)
        vec = jnp.where(lane128 == clsv, mc, vec)

        s_out = jnp.where(is_fbv, s0v, vv)
        offc = (clsv + 1).astype(jnp.float32) * OFF_STEP
        lbl = (clsv + 1).astype(jnp.float32)
        row = jnp.where(
            liota == 0, bx1 - offc,
            jnp.where(liota == 1, by1 - offc,
                      jnp.where(liota == 2, bx2 - offc,
                                jnp.where(liota == 3, by2 - offc,
                                          jnp.where(liota == 4, s_out, lbl)))))
        out_ref[pl.ds(t, 1), :] = row
        return (vec, mc, clsv, rest_n, cls_rn)

    rest0 = jnp.max(vec, keepdims=True)
    cls_r0 = jnp.min(jnp.where(vec == rest0, lane128, 127), keepdims=True)
    lax.fori_loop(0, NUM_DET, body,
                  (vec, jnp.full((1, 1), NEG_INF, jnp.float32),
                   jnp.zeros((1, 1), jnp.int32), rest0, cls_r0))


@jax.jit
def kernel(class_logit, box_regression, proposal):
    logit_t = jnp.pad(class_logit, ((0, NPAD - N), (0, 0))).T
    d4 = jnp.pad(
        jnp.transpose(box_regression.reshape(N, NUM_CLASSES, 4)[:, 1:, :],
                      (2, 1, 0)),
        ((0, 0), (0, 0), (0, NPAD - N)))
    prop_t = jnp.pad(proposal, ((0, NPAD - N), (0, 0))).T

    big = jax.ShapeDtypeStruct((C, NPAD), jnp.float32)
    pre = pl.pallas_call(
        _precompute_body,
        out_shape=(big,) * 5 + (jax.ShapeDtypeStruct((1, 1), jnp.float32),),
        interpret=_INTERPRET,
    )(logit_t, d4, prop_t)
    nbx1, nby1, nbx2, nby2, work, s0 = pre

    shaped = [a.reshape(C, NBLK, NSUB)
              for a in (nbx1, nby1, nbx2, nby2, work)]

    out = pl.pallas_call(
        _nms_body,
        out_shape=jax.ShapeDtypeStruct((NUM_DET, 8), jnp.float32),
        scratch_shapes=[pltpu.VMEM((C, NBLK, NSUB), jnp.float32)],
        interpret=_INTERPRET,
    )(*shaped, s0)

    boxes = out[:, 0:4]
    scores = out[:, 4]
    labels = out[:, 5].astype(jnp.int32)
    return boxes, scores, labels
